# Initial kernel scaffold; baseline (speedup 1.0000x reference)
import jax
import jax.numpy as jnp
from jax.experimental import pallas as pl


def kernel(users, items, user_emb, item_emb, edge_src, edge_dst, edge_w, fw1, fw2, fw3, fw4):
    raise NotImplementedError("write your pallas kernel here")


# speedup vs baseline: 7.4305x; 7.4305x over previous
import jax
import jax.numpy as jnp
from jax import lax
from jax.experimental import pallas as pl
from jax.experimental.pallas import tpu as pltpu
from jax.experimental.pallas import tpu_sc as plsc

NU = 25000
NN = 50000
NP = 50048
HALF = 32
E0 = 800000
BATCH = 4096
NC, NS = 2, 16
CH = 512
SUB = CH // 128
NCHUNK = 98
PT = NCHUNK * CH
EPAD = PT * NS
ZROWS = NP // NS
ZCH = 256
PB = BATCH // NS


def _body(x0_h, esrc_h, edst_h, ew_h, pidx_h, fw_h,
          gam_h, h1_h, h2_h, h3_h,
          src_v, dst_v, w_v, rows_v, zbuf, g_v, fw_v, acc, sem):
  c = lax.axis_index("c")
  s = lax.axis_index("s")

  zeros16 = jnp.zeros((16,), jnp.float32)

  @pl.loop(0, ZCH)
  def _zero_zbuf(e):
    zbuf[e, pl.ds(0, 16)] = zeros16
    zbuf[e, pl.ds(16, 16)] = zeros16

  hin = [x0_h, h1_h, h2_h]
  hout = [h1_h, h2_h, h3_h]
  nfull = ZROWS // ZCH
  rem = ZROWS % ZCH

  for l in range(3):
    zb = s * ZROWS
    for k in range(nfull):
      pltpu.sync_copy(zbuf, acc.at[pl.ds(zb + k * ZCH, ZCH)])
    if rem:
      pltpu.sync_copy(zbuf.at[pl.ds(0, rem)],
                      acc.at[pl.ds(zb + nfull * ZCH, rem)])
    plsc.subcore_barrier()

    src_tab = hin[l]

    @pl.loop(0, NCHUNK)
    def _edges(g):
      row0 = s * (PT // 128) + g * SUB
      pltpu.sync_copy(esrc_h.at[c, pl.ds(row0, SUB)], src_v)
      pltpu.sync_copy(edst_h.at[pl.ds(row0, SUB)], dst_v)
      pltpu.sync_copy(ew_h.at[pl.ds(s * PT + g * CH, CH)], w_v)
      descs = [
          pltpu.async_copy(src_tab.at[src_v.at[j]],
                           rows_v.at[pl.ds(j * 128, 128)], sem)
          for j in range(SUB)
      ]
      for d in descs:
        d.wait()

      @pl.loop(0, CH // 16)
      def _scale(gg):
        wv = w_v[pl.ds(gg * 16, 16)]
        for i in range(16):
          e = gg * 16 + i
          w = wv[i]
          rows_v[e, pl.ds(0, 16)] = rows_v[e, pl.ds(0, 16)] * w
          rows_v[e, pl.ds(16, 16)] = rows_v[e, pl.ds(16, 16)] * w

      for j in range(SUB):
        pltpu.sync_copy(rows_v.at[pl.ds(j * 128, 128)],
                        acc.at[dst_v.at[j]], add=True)

    plsc.subcore_barrier()

    ob = c * NP + zb
    for k in range(nfull):
      pltpu.sync_copy(acc.at[pl.ds(zb + k * ZCH, ZCH)],
                      hout[l].at[pl.ds(ob + k * ZCH, ZCH)])
    if rem:
      pltpu.sync_copy(acc.at[pl.ds(zb + nfull * ZCH, rem)],
                      hout[l].at[pl.ds(ob + nfull * ZCH, rem)])
    plsc.subcore_barrier()

  pltpu.sync_copy(fw_h, fw_v)
  pltpu.sync_copy(pidx_h.at[c, pl.ds(8 * s, 4)], src_v)

  hs = [x0_h, h1_h, h2_h, h3_h]
  for l in range(4):
    fwl = fw_v[l, pl.ds(0, 16)]
    for half in range(2):
      for j in range(2):
        pltpu.async_copy(hs[l].at[src_v.at[half * 2 + j]], g_v, sem).wait()
        base = half * PB + j * 128

        @pl.loop(0, 128)
        def _fuse(e, l=l, base=base, fwl=fwl):
          for h in (0, 16):
            v = fwl * g_v[e, pl.ds(h, 16)]
            if l == 0:
              rows_v[base + e, pl.ds(h, 16)] = v
            else:
              rows_v[base + e, pl.ds(h, 16)] = (
                  rows_v[base + e, pl.ds(h, 16)] + v)

  iota = lax.iota(jnp.int32, 16)

  @pl.loop(0, PB // 16)
  def _reduce(grp):
    accv = jnp.zeros((16,), jnp.float32)
    for i in range(16):
      e = grp * 16 + i
      v = (rows_v[e, pl.ds(0, 16)] * rows_v[PB + e, pl.ds(0, 16)]
           + rows_v[e, pl.ds(16, 16)] * rows_v[PB + e, pl.ds(16, 16)])
      sv = v[0]
      for q in range(1, 16):
        sv = sv + v[q]
      accv = accv + jnp.where(iota == i, sv, 0.0)
    w_v[pl.ds(grp * 16, 16)] = accv

  pltpu.sync_copy(w_v.at[pl.ds(0, PB)],
                  gam_h.at[pl.ds(c * BATCH + s * PB, PB)])


@jax.jit
def _run(x0, esrc2, edst2, ew, pidx, fwv):
  mesh = plsc.VectorSubcoreMesh(core_axis_name="c", subcore_axis_name="s",
                                num_cores=NC, num_subcores=NS)
  f = pl.kernel(
      _body,
      out_type=[
          jax.ShapeDtypeStruct((NC * BATCH,), jnp.float32),
          jax.ShapeDtypeStruct((NC * NP, HALF), jnp.float32),
          jax.ShapeDtypeStruct((NC * NP, HALF), jnp.float32),
          jax.ShapeDtypeStruct((NC * NP, HALF), jnp.float32),
      ],
      mesh=mesh,
      compiler_params=pltpu.CompilerParams(use_tc_tiling_on_sc=False),
      scratch_types=[
          pltpu.VMEM((SUB, 128), jnp.int32),
          pltpu.VMEM((SUB, 128), jnp.int32),
          pltpu.VMEM((CH,), jnp.float32),
          pltpu.VMEM((CH, HALF), jnp.float32),
          pltpu.VMEM((ZCH, HALF), jnp.float32),
          pltpu.VMEM((128, HALF), jnp.float32),
          pltpu.VMEM((8, 16), jnp.float32),
          pltpu.VMEM_SHARED((NP, HALF), jnp.float32),
          pltpu.SemaphoreType.DMA,
      ],
  )
  return f(x0, esrc2, edst2, ew, pidx, fwv)


def kernel(users, items, user_emb, item_emb, edge_src, edge_dst, edge_w,
           fw1, fw2, fw3, fw4):
  all_emb = jnp.concatenate([user_emb, item_emb], axis=0)
  npad = jnp.zeros((NP - NN, HALF), jnp.float32)
  x0 = jnp.concatenate(
      [all_emb[:, :HALF], npad, all_emb[:, HALF:], npad], 0)

  pad = EPAD - E0
  off = jnp.array([0, NP], jnp.int32).reshape(2, 1, 1)
  esrc = jnp.concatenate([edge_src.astype(jnp.int32),
                          jnp.zeros((pad,), jnp.int32)])
  esrc2 = esrc.reshape(1, EPAD // 128, 128) + off
  edst2 = jnp.concatenate(
      [edge_dst.astype(jnp.int32),
       jnp.zeros((pad,), jnp.int32)]).reshape(EPAD // 128, 128)
  ew = jnp.concatenate([edge_w.astype(jnp.float32),
                        jnp.zeros((pad,), jnp.float32)])

  u3 = users.astype(jnp.int32).reshape(NS, 2, 128)
  i3 = (items.astype(jnp.int32) + NU).reshape(NS, 2, 128)
  zp = jnp.zeros((NS, 4, 128), jnp.int32)
  pidx = (jnp.concatenate([u3, i3, zp], axis=1)
          .reshape(1, NS * 8, 128) + off)

  fwv = jnp.zeros((8, 16), jnp.float32)
  fwv = fwv.at[0:4].set(
      jnp.broadcast_to(
          jnp.stack([fw1, fw2, fw3, fw4]).reshape(4, 1).astype(jnp.float32),
          (4, 16)))

  gam, _, _, _ = _run(x0, esrc2, edst2, ew, pidx, fwv)
  return gam[:BATCH] + gam[BATCH:]


# speedup vs baseline: 7.7070x; 1.0372x over previous
import jax
import jax.numpy as jnp
from jax import lax
from jax.experimental import pallas as pl
from jax.experimental.pallas import tpu as pltpu
from jax.experimental.pallas import tpu_sc as plsc

NU = 25000
NN = 50000
NP = 50048
HALF = 32
E0 = 800000
BATCH = 4096
NC, NS = 2, 16
CH = 384
SUB = CH // 128
NCHUNK = 132
PT = NCHUNK * CH
EPAD = PT * NS
ROWB = PT // 128
ZROWS = NP // NS
PB = BATCH // NS


def _body(x0_h, esrc_h, edst_h, ew_h, pidx_h, fw_h,
          gam_h, h1_h, h2_h, h3_h,
          srcA, dstA, wA, rowsA, srcB, dstB, wB, rowsB,
          fw_v, acc, semA, semB):
  c = lax.axis_index("c")
  s = lax.axis_index("s")

  zeros16 = jnp.zeros((16,), jnp.float32)

  def stage_fire(g, srcb, dstb, wb, rowsb, sem, tab):
    row0 = s * ROWB + g * SUB
    pltpu.sync_copy(esrc_h.at[c, pl.ds(row0, SUB)], srcb)
    pltpu.sync_copy(edst_h.at[pl.ds(row0, SUB)], dstb)
    pltpu.sync_copy(ew_h.at[pl.ds(s * PT + g * CH, CH)], wb)
    for j in range(SUB):
      pltpu.async_copy(tab.at[srcb.at[j]],
                       rowsb.at[pl.ds(j * 128, 128)], sem)

  def wait_gathers(srcb, rowsb, sem, tab):
    for j in range(SUB):
      pltpu.make_async_copy(tab.at[srcb.at[j]],
                            rowsb.at[pl.ds(j * 128, 128)], sem).wait()

  def scale(wb, rowsb):
    @pl.loop(0, CH // 16)
    def _scale(gg):
      wv = wb[pl.ds(gg * 16, 16)]
      for i in range(16):
        e = gg * 16 + i
        w = wv[i]
        rowsb[e, pl.ds(0, 16)] = rowsb[e, pl.ds(0, 16)] * w
        rowsb[e, pl.ds(16, 16)] = rowsb[e, pl.ds(16, 16)] * w

  def scatter(dstb, rowsb):
    for j in range(SUB):
      pltpu.sync_copy(rowsb.at[pl.ds(j * 128, 128)],
                      acc.at[dstb.at[j]], add=True)

  hin = [x0_h, h1_h, h2_h]
  hout = [h1_h, h2_h, h3_h]

  for l in range(3):
    @pl.loop(0, CH)
    def _zero(e):
      rowsA[e, pl.ds(0, 16)] = zeros16
      rowsA[e, pl.ds(16, 16)] = zeros16

    zb = s * ZROWS
    for k in range(ZROWS // CH):
      pltpu.sync_copy(rowsA, acc.at[pl.ds(zb + k * CH, CH)])
    rem = ZROWS % CH
    if rem:
      pltpu.sync_copy(rowsA.at[pl.ds(0, rem)],
                      acc.at[pl.ds(zb + (ZROWS // CH) * CH, rem)])
    plsc.subcore_barrier()

    tab = hin[l]
    stage_fire(0, srcA, dstA, wA, rowsA, semA, tab)

    @pl.loop(0, NCHUNK // 2)
    def _edges(t):
      g1 = 2 * t + 1
      g2 = 2 * t + 2
      stage_fire(g1, srcB, dstB, wB, rowsB, semB, tab)
      wait_gathers(srcA, rowsA, semA, tab)
      scale(wA, rowsA)
      scatter(dstA, rowsA)
      g2w = jnp.where(g2 < NCHUNK, g2, 0)
      stage_fire(g2w, srcA, dstA, wA, rowsA, semA, tab)
      wait_gathers(srcB, rowsB, semB, tab)
      scale(wB, rowsB)
      scatter(dstB, rowsB)

    wait_gathers(srcA, rowsA, semA, tab)
    plsc.subcore_barrier()

    ob = c * NP + zb
    for k in range(ZROWS // CH):
      pltpu.sync_copy(acc.at[pl.ds(zb + k * CH, CH)],
                      hout[l].at[pl.ds(ob + k * CH, CH)])
    if rem:
      pltpu.sync_copy(acc.at[pl.ds(zb + (ZROWS // CH) * CH, rem)],
                      hout[l].at[pl.ds(ob + (ZROWS // CH) * CH, rem)])
    plsc.subcore_barrier()

  pltpu.sync_copy(fw_h, fw_v)
  pltpu.sync_copy(pidx_h.at[c, pl.ds(8 * s, 2)], srcA.at[pl.ds(0, 2)])
  pltpu.sync_copy(pidx_h.at[c, pl.ds(8 * s + 2, 2)], dstA.at[pl.ds(0, 2)])

  hs = [x0_h, h1_h, h2_h, h3_h]
  for l in range(4):
    fwl = fw_v[l, pl.ds(0, 16)]
    for half in range(2):
      idxb = srcA if half == 0 else dstA
      for j in range(2):
        pltpu.async_copy(hs[l].at[idxb.at[j]],
                         rowsB.at[pl.ds(0, 128)], semA).wait()

        @pl.loop(0, 128)
        def _fuse(e, l=l, half=half, j=j, fwl=fwl):
          for h in (0, 16):
            v = fwl * rowsB[e, pl.ds(h, 16)]
            if half == 0:
              tgt, row = rowsA, j * 128 + e
            else:
              tgt, row = rowsB, 128 + j * 128 + e
            if l == 0:
              tgt[row, pl.ds(h, 16)] = v
            else:
              tgt[row, pl.ds(h, 16)] = tgt[row, pl.ds(h, 16)] + v

  iota = lax.iota(jnp.int32, 16)

  @pl.loop(0, PB // 16)
  def _reduce(grp):
    accv = jnp.zeros((16,), jnp.float32)
    for i in range(16):
      e = grp * 16 + i
      v = (rowsA[e, pl.ds(0, 16)] * rowsB[128 + e, pl.ds(0, 16)]
           + rowsA[e, pl.ds(16, 16)] * rowsB[128 + e, pl.ds(16, 16)])
      sv = v[0]
      for q in range(1, 16):
        sv = sv + v[q]
      accv = accv + jnp.where(iota == i, sv, 0.0)
    wA[pl.ds(grp * 16, 16)] = accv

  pltpu.sync_copy(wA.at[pl.ds(0, PB)],
                  gam_h.at[pl.ds(c * BATCH + s * PB, PB)])


@jax.jit
def _run(x0, esrc2, edst2, ew, pidx, fwv):
  mesh = plsc.VectorSubcoreMesh(core_axis_name="c", subcore_axis_name="s",
                                num_cores=NC, num_subcores=NS)
  f = pl.kernel(
      _body,
      out_type=[
          jax.ShapeDtypeStruct((NC * BATCH,), jnp.float32),
          jax.ShapeDtypeStruct((NC * NP, HALF), jnp.float32),
          jax.ShapeDtypeStruct((NC * NP, HALF), jnp.float32),
          jax.ShapeDtypeStruct((NC * NP, HALF), jnp.float32),
      ],
      mesh=mesh,
      compiler_params=pltpu.CompilerParams(use_tc_tiling_on_sc=False),
      scratch_types=[
          pltpu.VMEM((SUB, 128), jnp.int32),
          pltpu.VMEM((SUB, 128), jnp.int32),
          pltpu.VMEM((CH,), jnp.float32),
          pltpu.VMEM((CH, HALF), jnp.float32),
          pltpu.VMEM((SUB, 128), jnp.int32),
          pltpu.VMEM((SUB, 128), jnp.int32),
          pltpu.VMEM((CH,), jnp.float32),
          pltpu.VMEM((CH, HALF), jnp.float32),
          pltpu.VMEM((8, 16), jnp.float32),
          pltpu.VMEM_SHARED((NP, HALF), jnp.float32),
          pltpu.SemaphoreType.DMA,
          pltpu.SemaphoreType.DMA,
      ],
  )
  return f(x0, esrc2, edst2, ew, pidx, fwv)


def kernel(users, items, user_emb, item_emb, edge_src, edge_dst, edge_w,
           fw1, fw2, fw3, fw4):
  all_emb = jnp.concatenate([user_emb, item_emb], axis=0)
  npad = jnp.zeros((NP - NN, HALF), jnp.float32)
  x0 = jnp.concatenate(
      [all_emb[:, :HALF], npad, all_emb[:, HALF:], npad], 0)

  pad = EPAD - E0
  off = jnp.array([0, NP], jnp.int32).reshape(2, 1, 1)
  esrc = jnp.concatenate([edge_src.astype(jnp.int32),
                          jnp.zeros((pad,), jnp.int32)])
  esrc2 = esrc.reshape(1, EPAD // 128, 128) + off
  edst2 = jnp.concatenate(
      [edge_dst.astype(jnp.int32),
       jnp.zeros((pad,), jnp.int32)]).reshape(EPAD // 128, 128)
  ew = jnp.concatenate([edge_w.astype(jnp.float32),
                        jnp.zeros((pad,), jnp.float32)])

  u3 = users.astype(jnp.int32).reshape(NS, 2, 128)
  i3 = (items.astype(jnp.int32) + NU).reshape(NS, 2, 128)
  zp = jnp.zeros((NS, 4, 128), jnp.int32)
  pidx = (jnp.concatenate([u3, i3, zp], axis=1)
          .reshape(1, NS * 8, 128) + off)

  fwv = jnp.zeros((8, 16), jnp.float32)
  fwv = fwv.at[0:4].set(
      jnp.broadcast_to(
          jnp.stack([fw1, fw2, fw3, fw4]).reshape(4, 1).astype(jnp.float32),
          (4, 16)))

  gam, _, _, _ = _run(x0, esrc2, edst2, ew, pidx, fwv)
  return gam[:BATCH] + gam[BATCH:]


# speedup vs baseline: 8.8490x; 1.1482x over previous
import jax
import jax.numpy as jnp
from jax import lax
from jax.experimental import pallas as pl
from jax.experimental.pallas import tpu as pltpu
from jax.experimental.pallas import tpu_sc as plsc

NU = 25000
NN = 50000
NP = 50048
HALF = 32
E0 = 800000
BATCH = 4096
NC, NS = 2, 16
CH = 384
NCHUNK = 132
PT = NCHUNK * CH
EPAD = PT * NS
ZROWS = NP // NS
PB = BATCH // NS


def _body(x0_h, epk_h, ew_h, pidx_h, fw_h,
          gam_h, h1_h, h2_h, h3_h,
          ebA, wA, rowsA, ebB, wB, rowsB, fw_v, acc, semA, semB):
  c = lax.axis_index("c")
  s = lax.axis_index("s")

  zeros16 = jnp.zeros((16,), jnp.float32)

  def stage_fire(g, eb, wb, rowsb, sem, tab):
    pltpu.sync_copy(epk_h.at[c, s * NCHUNK + g], eb)
    pltpu.sync_copy(ew_h.at[pl.ds(s * PT + g * CH, CH)], wb)
    pltpu.async_copy(tab.at[eb.at[0]], rowsb, sem)

  def wait_gather(eb, rowsb, sem, tab):
    pltpu.make_async_copy(tab.at[eb.at[0]], rowsb, sem).wait()

  def scale(wb, rowsb):
    @pl.loop(0, CH // 16)
    def _scale(gg):
      wv = wb[pl.ds(gg * 16, 16)]
      for i in range(16):
        e = gg * 16 + i
        w = wv[i]
        rowsb[e, pl.ds(0, 16)] = rowsb[e, pl.ds(0, 16)] * w
        rowsb[e, pl.ds(16, 16)] = rowsb[e, pl.ds(16, 16)] * w

  def scatter(eb, rowsb):
    pltpu.sync_copy(rowsb, acc.at[eb.at[1]], add=True)

  hin = [x0_h, h1_h, h2_h]
  hout = [h1_h, h2_h, h3_h]

  for l in range(3):
    @pl.loop(0, CH)
    def _zero(e):
      rowsA[e, pl.ds(0, 16)] = zeros16
      rowsA[e, pl.ds(16, 16)] = zeros16

    zb = s * ZROWS
    for k in range(ZROWS // CH):
      pltpu.sync_copy(rowsA, acc.at[pl.ds(zb + k * CH, CH)])
    rem = ZROWS % CH
    if rem:
      pltpu.sync_copy(rowsA.at[pl.ds(0, rem)],
                      acc.at[pl.ds(zb + (ZROWS // CH) * CH, rem)])
    plsc.subcore_barrier()

    tab = hin[l]
    stage_fire(0, ebA, wA, rowsA, semA, tab)

    @pl.loop(0, NCHUNK // 2)
    def _edges(t):
      g1 = 2 * t + 1
      g2 = 2 * t + 2
      stage_fire(g1, ebB, wB, rowsB, semB, tab)
      wait_gather(ebA, rowsA, semA, tab)
      scale(wA, rowsA)
      scatter(ebA, rowsA)
      g2w = jnp.where(g2 < NCHUNK, g2, 0)
      stage_fire(g2w, ebA, wA, rowsA, semA, tab)
      wait_gather(ebB, rowsB, semB, tab)
      scale(wB, rowsB)
      scatter(ebB, rowsB)

    wait_gather(ebA, rowsA, semA, tab)
    plsc.subcore_barrier()

    ob = c * NP + zb
    for k in range(ZROWS // CH):
      pltpu.sync_copy(acc.at[pl.ds(zb + k * CH, CH)],
                      hout[l].at[pl.ds(ob + k * CH, CH)])
    if rem:
      pltpu.sync_copy(acc.at[pl.ds(zb + (ZROWS // CH) * CH, rem)],
                      hout[l].at[pl.ds(ob + (ZROWS // CH) * CH, rem)])
    plsc.subcore_barrier()

  pltpu.sync_copy(fw_h, fw_v)
  pltpu.sync_copy(pidx_h.at[c, pl.ds(2 * s, 2)], ebA.at[pl.ds(0, 2)])

  hs = [x0_h, h1_h, h2_h, h3_h]
  for l in range(4):
    fwl = fw_v[l, pl.ds(0, 16)]
    for half in range(2):
      for j in range(2):
        pltpu.async_copy(hs[l].at[ebA.at[half, pl.ds(j * 128, 128)]],
                         rowsB.at[pl.ds(0, 128)], semA).wait()

        @pl.loop(0, 128)
        def _fuse(e, l=l, half=half, j=j, fwl=fwl):
          for h in (0, 16):
            v = fwl * rowsB[e, pl.ds(h, 16)]
            if half == 0:
              tgt, row = rowsA, j * 128 + e
            else:
              tgt, row = rowsB, 128 + j * 128 + e
            if l == 0:
              tgt[row, pl.ds(h, 16)] = v
            else:
              tgt[row, pl.ds(h, 16)] = tgt[row, pl.ds(h, 16)] + v

  iota = lax.iota(jnp.int32, 16)

  @pl.loop(0, PB // 16)
  def _reduce(grp):
    accv = jnp.zeros((16,), jnp.float32)
    for i in range(16):
      e = grp * 16 + i
      v = (rowsA[e, pl.ds(0, 16)] * rowsB[128 + e, pl.ds(0, 16)]
           + rowsA[e, pl.ds(16, 16)] * rowsB[128 + e, pl.ds(16, 16)])
      sv = v[0]
      for q in range(1, 16):
        sv = sv + v[q]
      accv = accv + jnp.where(iota == i, sv, 0.0)
    rowsA[grp // 2, pl.ds((grp % 2) * 16, 16)] = accv

  pltpu.sync_copy(rowsA.at[pl.ds(0, PB // HALF)],
                  gam_h.at[pl.ds((c * NS + s) * (PB // HALF), PB // HALF)])


@jax.jit
def _run(x0, epk, ew, pidx, fwv):
  mesh = plsc.VectorSubcoreMesh(core_axis_name="c", subcore_axis_name="s",
                                num_cores=NC, num_subcores=NS)
  f = pl.kernel(
      _body,
      out_type=[
          jax.ShapeDtypeStruct((NC * BATCH // HALF, HALF), jnp.float32),
          jax.ShapeDtypeStruct((NC * NP, HALF), jnp.float32),
          jax.ShapeDtypeStruct((NC * NP, HALF), jnp.float32),
          jax.ShapeDtypeStruct((NC * NP, HALF), jnp.float32),
      ],
      mesh=mesh,
      compiler_params=pltpu.CompilerParams(use_tc_tiling_on_sc=False),
      scratch_types=[
          pltpu.VMEM((2, CH), jnp.int32),
          pltpu.VMEM((CH,), jnp.float32),
          pltpu.VMEM((CH, HALF), jnp.float32),
          pltpu.VMEM((2, CH), jnp.int32),
          pltpu.VMEM((CH,), jnp.float32),
          pltpu.VMEM((CH, HALF), jnp.float32),
          pltpu.VMEM((8, 16), jnp.float32),
          pltpu.VMEM_SHARED((NP, HALF), jnp.float32),
          pltpu.SemaphoreType.DMA,
          pltpu.SemaphoreType.DMA,
      ],
  )
  return f(x0, epk, ew, pidx, fwv)


def kernel(users, items, user_emb, item_emb, edge_src, edge_dst, edge_w,
           fw1, fw2, fw3, fw4):
  all_emb = jnp.concatenate([user_emb, item_emb], axis=0)
  npad = jnp.zeros((NP - NN, HALF), jnp.float32)
  x0 = jnp.concatenate(
      [all_emb[:, :HALF], npad, all_emb[:, HALF:], npad], 0)

  pad = EPAD - E0
  esrc = jnp.concatenate([edge_src.astype(jnp.int32),
                          jnp.zeros((pad,), jnp.int32)]).reshape(
                              NS * NCHUNK, CH)
  edst = jnp.concatenate([edge_dst.astype(jnp.int32),
                          jnp.zeros((pad,), jnp.int32)]).reshape(
                              NS * NCHUNK, CH)
  ew = jnp.concatenate([edge_w.astype(jnp.float32),
                        jnp.zeros((pad,), jnp.float32)])
  epk = jnp.stack([
      jnp.stack([esrc, edst], axis=1),
      jnp.stack([esrc + NP, edst], axis=1),
  ], axis=0)

  zpad = jnp.zeros((NS, CH - PB), jnp.int32)
  u2 = jnp.concatenate([users.astype(jnp.int32).reshape(NS, PB), zpad], 1)
  i2 = jnp.concatenate([(items.astype(jnp.int32) + NU).reshape(NS, PB),
                        zpad], 1)
  pidx0 = jnp.stack([u2, i2], axis=1).reshape(NS * 2, CH)
  pidx = jnp.stack([pidx0, pidx0 + NP], axis=0)

  fwv = jnp.zeros((8, 16), jnp.float32)
  fwv = fwv.at[0:4].set(
      jnp.broadcast_to(
          jnp.stack([fw1, fw2, fw3, fw4]).reshape(4, 1).astype(jnp.float32),
          (4, 16)))

  gam, _, _, _ = _run(x0, epk, ew, pidx, fwv)
  gam = gam.reshape(NC, BATCH)
  return gam[0] + gam[1]


# speedup vs baseline: 11.1771x; 1.2631x over previous
import jax
import jax.numpy as jnp
from jax import lax
from jax.experimental import pallas as pl
from jax.experimental.pallas import tpu as pltpu
from jax.experimental.pallas import tpu_sc as plsc

NU = 25000
NN = 50000
NP = 50048
HALF = 32
E0 = 800000
BATCH = 4096
NC, NS = 2, 16
CH = 384
NCHUNK = 132
PT = NCHUNK * CH
EPAD = PT * NS
ZROWS = NP // NS
ZFULL = ZROWS // CH
ZREM = ZROWS % CH
PB = BATCH // NS


def _body(x0_h, epk_h, ew_h, pidx_h, fw_h,
          gam_h, h1_h, h2_h, h3_h,
          eb0, eb1, eb2, eb3, w0, w1, w2, w3, rows0, rows1,
          fw_v, acc, semA, semB, semI0, semI1, semI2, semI3, semZ):
  c = lax.axis_index("c")
  s = lax.axis_index("s")

  ebs = [eb0, eb1, eb2, eb3]
  ws = [w0, w1, w2, w3]
  semIs = [semI0, semI1, semI2, semI3]
  zeros16 = jnp.zeros((16,), jnp.float32)

  def pf_idx(g, k):
    gg = jnp.where(g < NCHUNK, g, 0)
    pltpu.async_copy(epk_h.at[c, s * NCHUNK + gg], ebs[k], semIs[k])
    pltpu.async_copy(ew_h.at[pl.ds(s * PT + gg * CH, CH)], ws[k], semIs[k])

  def wt_idx(k):
    pltpu.make_async_copy(epk_h.at[c, 0], ebs[k], semIs[k]).wait()
    pltpu.make_async_copy(ew_h.at[pl.ds(0, CH)], ws[k], semIs[k]).wait()

  def fire_gather(k, rowsb, sem, tab):
    pltpu.async_copy(tab.at[ebs[k].at[0]], rowsb, sem)

  def wt_gather(k, rowsb, sem, tab):
    pltpu.make_async_copy(tab.at[ebs[k].at[0]], rowsb, sem).wait()

  def scale(k, rowsb):
    wb = ws[k]

    @pl.loop(0, CH // 16)
    def _scale(gg):
      wv = wb[pl.ds(gg * 16, 16)]
      for i in range(16):
        e = gg * 16 + i
        w = wv[i]
        rowsb[e, pl.ds(0, 16)] = rowsb[e, pl.ds(0, 16)] * w
        rowsb[e, pl.ds(16, 16)] = rowsb[e, pl.ds(16, 16)] * w

  def scatter(k, rowsb):
    pltpu.sync_copy(rowsb, acc.at[ebs[k].at[1]], add=True)

  hin = [x0_h, h1_h, h2_h]
  hout = [h1_h, h2_h, h3_h]

  for l in range(3):
    @pl.loop(0, CH)
    def _zero(e):
      rows0[e, pl.ds(0, 16)] = zeros16
      rows0[e, pl.ds(16, 16)] = zeros16

    zb = s * ZROWS
    for k in range(ZFULL):
      pltpu.async_copy(rows0, acc.at[pl.ds(zb + k * CH, CH)], semZ)
    pltpu.async_copy(rows0.at[pl.ds(0, ZREM)],
                     acc.at[pl.ds(zb + ZFULL * CH, ZREM)], semZ)
    for k in range(ZFULL):
      pltpu.make_async_copy(rows0, acc.at[pl.ds(zb + k * CH, CH)],
                            semZ).wait()
    pltpu.make_async_copy(rows0.at[pl.ds(0, ZREM)],
                          acc.at[pl.ds(zb + ZFULL * CH, ZREM)], semZ).wait()
    plsc.subcore_barrier()

    tab = hin[l]
    for k in range(4):
      pf_idx(jnp.int32(k), k)
    wt_idx(0)
    fire_gather(0, rows0, semA, tab)
    wt_idx(1)
    fire_gather(1, rows1, semB, tab)

    @pl.loop(0, NCHUNK // 4)
    def _edges(u):
      g = 4 * u
      wt_gather(0, rows0, semA, tab)
      scale(0, rows0)
      scatter(0, rows0)
      wt_idx(2)
      fire_gather(2, rows0, semA, tab)
      pf_idx(g + 4, 0)
      wt_gather(1, rows1, semB, tab)
      scale(1, rows1)
      scatter(1, rows1)
      wt_idx(3)
      fire_gather(3, rows1, semB, tab)
      pf_idx(g + 5, 1)
      wt_gather(2, rows0, semA, tab)
      scale(2, rows0)
      scatter(2, rows0)
      wt_idx(0)
      fire_gather(0, rows0, semA, tab)
      pf_idx(g + 6, 2)
      wt_gather(3, rows1, semB, tab)
      scale(3, rows1)
      scatter(3, rows1)
      wt_idx(1)
      fire_gather(1, rows1, semB, tab)
      pf_idx(g + 7, 3)

    wt_gather(0, rows0, semA, tab)
    wt_gather(1, rows1, semB, tab)
    wt_idx(2)
    wt_idx(3)
    plsc.subcore_barrier()

    ob = c * NP + zb
    for k in range(ZFULL):
      pltpu.async_copy(acc.at[pl.ds(zb + k * CH, CH)],
                       hout[l].at[pl.ds(ob + k * CH, CH)], semZ)
    pltpu.async_copy(acc.at[pl.ds(zb + ZFULL * CH, ZREM)],
                     hout[l].at[pl.ds(ob + ZFULL * CH, ZREM)], semZ)
    for k in range(ZFULL):
      pltpu.make_async_copy(acc.at[pl.ds(zb + k * CH, CH)],
                            hout[l].at[pl.ds(ob + k * CH, CH)], semZ).wait()
    pltpu.make_async_copy(acc.at[pl.ds(zb + ZFULL * CH, ZREM)],
                          hout[l].at[pl.ds(ob + ZFULL * CH, ZREM)],
                          semZ).wait()
    plsc.subcore_barrier()

  pltpu.sync_copy(fw_h, fw_v)
  pltpu.sync_copy(pidx_h.at[c, pl.ds(2 * s, 2)], eb0)

  hs = [x0_h, h1_h, h2_h, h3_h]
  for l in range(4):
    fwl = fw_v[l, pl.ds(0, 16)]
    for half in range(2):
      for j in range(2):
        pltpu.async_copy(hs[l].at[eb0.at[half, pl.ds(j * 128, 128)]],
                         rows1.at[pl.ds(0, 128)], semA).wait()

        @pl.loop(0, 128)
        def _fuse(e, l=l, half=half, j=j, fwl=fwl):
          for h in (0, 16):
            v = fwl * rows1[e, pl.ds(h, 16)]
            if half == 0:
              tgt, row = rows0, j * 128 + e
            else:
              tgt, row = rows1, 128 + j * 128 + e
            if l == 0:
              tgt[row, pl.ds(h, 16)] = v
            else:
              tgt[row, pl.ds(h, 16)] = tgt[row, pl.ds(h, 16)] + v

  iota = lax.iota(jnp.int32, 16)

  @pl.loop(0, PB // 16)
  def _reduce(grp):
    accv = jnp.zeros((16,), jnp.float32)
    for i in range(16):
      e = grp * 16 + i
      v = (rows0[e, pl.ds(0, 16)] * rows1[128 + e, pl.ds(0, 16)]
           + rows0[e, pl.ds(16, 16)] * rows1[128 + e, pl.ds(16, 16)])
      sv = v[0]
      for q in range(1, 16):
        sv = sv + v[q]
      accv = accv + jnp.where(iota == i, sv, 0.0)
    rows0[grp // 2, pl.ds((grp % 2) * 16, 16)] = accv

  pltpu.sync_copy(rows0.at[pl.ds(0, PB // HALF)],
                  gam_h.at[pl.ds((c * NS + s) * (PB // HALF), PB // HALF)])


@jax.jit
def _run(x0, epk, ew, pidx, fwv):
  mesh = plsc.VectorSubcoreMesh(core_axis_name="c", subcore_axis_name="s",
                                num_cores=NC, num_subcores=NS)
  f = pl.kernel(
      _body,
      out_type=[
          jax.ShapeDtypeStruct((NC * BATCH // HALF, HALF), jnp.float32),
          jax.ShapeDtypeStruct((NC * NP, HALF), jnp.float32),
          jax.ShapeDtypeStruct((NC * NP, HALF), jnp.float32),
          jax.ShapeDtypeStruct((NC * NP, HALF), jnp.float32),
      ],
      mesh=mesh,
      compiler_params=pltpu.CompilerParams(use_tc_tiling_on_sc=False),
      scratch_types=[
          pltpu.VMEM((2, CH), jnp.int32),
          pltpu.VMEM((2, CH), jnp.int32),
          pltpu.VMEM((2, CH), jnp.int32),
          pltpu.VMEM((2, CH), jnp.int32),
          pltpu.VMEM((CH,), jnp.float32),
          pltpu.VMEM((CH,), jnp.float32),
          pltpu.VMEM((CH,), jnp.float32),
          pltpu.VMEM((CH,), jnp.float32),
          pltpu.VMEM((CH, HALF), jnp.float32),
          pltpu.VMEM((CH, HALF), jnp.float32),
          pltpu.VMEM((8, 16), jnp.float32),
          pltpu.VMEM_SHARED((NP, HALF), jnp.float32),
          pltpu.SemaphoreType.DMA,
          pltpu.SemaphoreType.DMA,
          pltpu.SemaphoreType.DMA,
          pltpu.SemaphoreType.DMA,
          pltpu.SemaphoreType.DMA,
          pltpu.SemaphoreType.DMA,
          pltpu.SemaphoreType.DMA,
      ],
  )
  return f(x0, epk, ew, pidx, fwv)


def kernel(users, items, user_emb, item_emb, edge_src, edge_dst, edge_w,
           fw1, fw2, fw3, fw4):
  all_emb = jnp.concatenate([user_emb, item_emb], axis=0)
  npad = jnp.zeros((NP - NN, HALF), jnp.float32)
  x0 = jnp.concatenate(
      [all_emb[:, :HALF], npad, all_emb[:, HALF:], npad], 0)

  pad = EPAD - E0
  esrc = jnp.concatenate([edge_src.astype(jnp.int32),
                          jnp.zeros((pad,), jnp.int32)]).reshape(
                              NS * NCHUNK, CH)
  edst = jnp.concatenate([edge_dst.astype(jnp.int32),
                          jnp.zeros((pad,), jnp.int32)]).reshape(
                              NS * NCHUNK, CH)
  ew = jnp.concatenate([edge_w.astype(jnp.float32),
                        jnp.zeros((pad,), jnp.float32)])
  epk = jnp.stack([
      jnp.stack([esrc, edst], axis=1),
      jnp.stack([esrc + NP, edst], axis=1),
  ], axis=0)

  zpad = jnp.zeros((NS, CH - PB), jnp.int32)
  u2 = jnp.concatenate([users.astype(jnp.int32).reshape(NS, PB), zpad], 1)
  i2 = jnp.concatenate([(items.astype(jnp.int32) + NU).reshape(NS, PB),
                        zpad], 1)
  pidx0 = jnp.stack([u2, i2], axis=1).reshape(NS * 2, CH)
  pidx = jnp.stack([pidx0, pidx0 + NP], axis=0)

  fwv = jnp.zeros((8, 16), jnp.float32)
  fwv = fwv.at[0:4].set(
      jnp.broadcast_to(
          jnp.stack([fw1, fw2, fw3, fw4]).reshape(4, 1).astype(jnp.float32),
          (4, 16)))

  gam, _, _, _ = _run(x0, epk, ew, pidx, fwv)
  gam = gam.reshape(NC, BATCH)
  return gam[0] + gam[1]


# speedup vs baseline: 11.2931x; 1.0104x over previous
import jax
import jax.numpy as jnp
from jax import lax
from jax.experimental import pallas as pl
from jax.experimental.pallas import tpu as pltpu
from jax.experimental.pallas import tpu_sc as plsc

NU = 25000
NN = 50000
NP = 50048
HALF = 32
E0 = 800000
BATCH = 4096
NC, NS = 2, 16
CH = 384
NCHUNK = 132
PT = NCHUNK * CH
EPAD = PT * NS
ZROWS = NP // NS
ZFULL = ZROWS // CH
ZREM = ZROWS % CH
PB = BATCH // NS


def _body(x0_h, epk_h, ew_h, pidx_h, fw_h,
          gam_h, h1_h, h2_h, h3_h,
          eb0, eb1, eb2, eb3, w0, w1, w2, w3, rows0, rows1,
          fw_v, acc, semA, semB, semI0, semI1, semI2, semI3, semZ):
  c = lax.axis_index("c")
  s = lax.axis_index("s")

  ebs = [eb0, eb1, eb2, eb3]
  ws = [w0, w1, w2, w3]
  semIs = [semI0, semI1, semI2, semI3]
  zeros16 = jnp.zeros((16,), jnp.float32)

  def pf_idx(g, k):
    gg = jnp.where(g < NCHUNK, g, 0)
    pltpu.async_copy(epk_h.at[c, s * NCHUNK + gg], ebs[k], semIs[k])
    pltpu.async_copy(ew_h.at[pl.ds(s * PT + gg * CH, CH)], ws[k], semIs[k])

  def wt_idx(k):
    pltpu.make_async_copy(epk_h.at[c, 0], ebs[k], semIs[k]).wait()
    pltpu.make_async_copy(ew_h.at[pl.ds(0, CH)], ws[k], semIs[k]).wait()

  def fire_gather(k, rowsb, sem, tab):
    pltpu.async_copy(tab.at[ebs[k].at[0]], rowsb, sem)

  def wt_gather(k, rowsb, sem, tab):
    pltpu.make_async_copy(tab.at[ebs[k].at[0]], rowsb, sem).wait()

  def scale(k, rowsb):
    wb = ws[k]

    @pl.loop(0, CH // 16)
    def _scale(gg):
      wv = wb[pl.ds(gg * 16, 16)]
      for i in range(16):
        e = gg * 16 + i
        w = wv[i]
        rowsb[e, pl.ds(0, 16)] = rowsb[e, pl.ds(0, 16)] * w
        rowsb[e, pl.ds(16, 16)] = rowsb[e, pl.ds(16, 16)] * w

  def scatter(k, rowsb):
    pltpu.sync_copy(rowsb, acc.at[ebs[k].at[1]], add=True)

  hin = [x0_h, h1_h, h2_h]
  hout = [h1_h, h2_h, h3_h]

  zb = s * ZROWS
  ob0 = zb

  for k in range(4):
    pf_idx(jnp.int32(k), k)

  for l in range(3):
    @pl.loop(0, CH)
    def _zero(e):
      rows0[e, pl.ds(0, 16)] = zeros16
      rows0[e, pl.ds(16, 16)] = zeros16

    for k in range(ZFULL):
      pltpu.async_copy(rows0, acc.at[pl.ds(zb + k * CH, CH)], semZ)
    pltpu.async_copy(rows0.at[pl.ds(0, ZREM)],
                     acc.at[pl.ds(zb + ZFULL * CH, ZREM)], semZ)
    for k in range(ZFULL):
      pltpu.make_async_copy(rows0, acc.at[pl.ds(zb + k * CH, CH)],
                            semZ).wait()
    pltpu.make_async_copy(rows0.at[pl.ds(0, ZREM)],
                          acc.at[pl.ds(zb + ZFULL * CH, ZREM)], semZ).wait()
    plsc.subcore_barrier()

    tab = hin[l]
    wt_idx(0)
    fire_gather(0, rows0, semA, tab)
    wt_idx(1)
    fire_gather(1, rows1, semB, tab)

    @pl.loop(0, NCHUNK // 4)
    def _edges(u):
      g = 4 * u
      wt_gather(0, rows0, semA, tab)
      scale(0, rows0)
      scatter(0, rows0)
      wt_idx(2)
      fire_gather(2, rows0, semA, tab)
      pf_idx(g + 4, 0)
      wt_gather(1, rows1, semB, tab)
      scale(1, rows1)
      scatter(1, rows1)
      wt_idx(3)
      fire_gather(3, rows1, semB, tab)
      pf_idx(g + 5, 1)
      wt_gather(2, rows0, semA, tab)
      scale(2, rows0)
      scatter(2, rows0)
      wt_idx(0)
      fire_gather(0, rows0, semA, tab)
      pf_idx(g + 6, 2)
      wt_gather(3, rows1, semB, tab)
      scale(3, rows1)
      scatter(3, rows1)
      wt_idx(1)
      fire_gather(1, rows1, semB, tab)
      pf_idx(g + 7, 3)

    wt_gather(0, rows0, semA, tab)
    wt_gather(1, rows1, semB, tab)
    wt_idx(2)
    wt_idx(3)
    plsc.subcore_barrier()

    for k in range(4):
      pf_idx(jnp.int32(k), k)

    ob = c * NP + zb
    for k in range(ZFULL):
      pltpu.async_copy(acc.at[pl.ds(zb + k * CH, CH)],
                       hout[l].at[pl.ds(ob + k * CH, CH)], semZ)
    pltpu.async_copy(acc.at[pl.ds(zb + ZFULL * CH, ZREM)],
                     hout[l].at[pl.ds(ob + ZFULL * CH, ZREM)], semZ)
    for k in range(ZFULL):
      pltpu.make_async_copy(acc.at[pl.ds(zb + k * CH, CH)],
                            hout[l].at[pl.ds(ob + k * CH, CH)], semZ).wait()
    pltpu.make_async_copy(acc.at[pl.ds(zb + ZFULL * CH, ZREM)],
                          hout[l].at[pl.ds(ob + ZFULL * CH, ZREM)],
                          semZ).wait()

  for k in range(4):
    wt_idx(k)

  pltpu.sync_copy(fw_h, fw_v)
  pltpu.sync_copy(pidx_h.at[c, pl.ds(2 * s, 2)], eb0)

  hs = [x0_h, h1_h, h2_h, h3_h]
  steps = [(l, half, j) for l in range(4) for half in range(2)
           for j in range(2)]

  def _stage_ref(n):
    if n % 2 == 0:
      return rows1.at[pl.ds(0, 128)], semA
    return rows0.at[pl.ds(256, 128)], semB

  def _fire_final(n):
    l, half, j = steps[n]
    ref, sem = _stage_ref(n)
    pltpu.async_copy(hs[l].at[eb0.at[half, pl.ds(j * 128, 128)]], ref, sem)

  def _wait_final(n):
    l, half, j = steps[n]
    ref, sem = _stage_ref(n)
    pltpu.make_async_copy(hs[l].at[eb0.at[half, pl.ds(j * 128, 128)]],
                          ref, sem).wait()

  _fire_final(0)
  for n in range(16):
    _wait_final(n)
    if n + 1 < 16:
      _fire_final(n + 1)
    l, half, j = steps[n]
    fwl = fw_v[l, pl.ds(0, 16)]
    stg = rows1 if n % 2 == 0 else rows0
    srow = 0 if n % 2 == 0 else 256

    @pl.loop(0, 128)
    def _fuse(e, l=l, half=half, j=j, fwl=fwl, stg=stg, srow=srow):
      for h in (0, 16):
        v = fwl * stg[srow + e, pl.ds(h, 16)]
        if half == 0:
          tgt, row = rows0, j * 128 + e
        else:
          tgt, row = rows1, 128 + j * 128 + e
        if l == 0:
          tgt[row, pl.ds(h, 16)] = v
        else:
          tgt[row, pl.ds(h, 16)] = tgt[row, pl.ds(h, 16)] + v

  iota = lax.iota(jnp.int32, 16)

  @pl.loop(0, PB // 16)
  def _reduce(grp):
    accv = jnp.zeros((16,), jnp.float32)
    for i in range(16):
      e = grp * 16 + i
      v = (rows0[e, pl.ds(0, 16)] * rows1[128 + e, pl.ds(0, 16)]
           + rows0[e, pl.ds(16, 16)] * rows1[128 + e, pl.ds(16, 16)])
      sv = v[0]
      for q in range(1, 16):
        sv = sv + v[q]
      accv = accv + jnp.where(iota == i, sv, 0.0)
    rows0[grp // 2, pl.ds((grp % 2) * 16, 16)] = accv

  pltpu.sync_copy(rows0.at[pl.ds(0, PB // HALF)],
                  gam_h.at[pl.ds((c * NS + s) * (PB // HALF), PB // HALF)])


@jax.jit
def _run(x0, epk, ew, pidx, fwv):
  mesh = plsc.VectorSubcoreMesh(core_axis_name="c", subcore_axis_name="s",
                                num_cores=NC, num_subcores=NS)
  f = pl.kernel(
      _body,
      out_type=[
          jax.ShapeDtypeStruct((NC * BATCH // HALF, HALF), jnp.float32),
          jax.ShapeDtypeStruct((NC * NP, HALF), jnp.float32),
          jax.ShapeDtypeStruct((NC * NP, HALF), jnp.float32),
          jax.ShapeDtypeStruct((NC * NP, HALF), jnp.float32),
      ],
      mesh=mesh,
      compiler_params=pltpu.CompilerParams(use_tc_tiling_on_sc=False),
      scratch_types=[
          pltpu.VMEM((2, CH), jnp.int32),
          pltpu.VMEM((2, CH), jnp.int32),
          pltpu.VMEM((2, CH), jnp.int32),
          pltpu.VMEM((2, CH), jnp.int32),
          pltpu.VMEM((CH,), jnp.float32),
          pltpu.VMEM((CH,), jnp.float32),
          pltpu.VMEM((CH,), jnp.float32),
          pltpu.VMEM((CH,), jnp.float32),
          pltpu.VMEM((CH, HALF), jnp.float32),
          pltpu.VMEM((CH, HALF), jnp.float32),
          pltpu.VMEM((8, 16), jnp.float32),
          pltpu.VMEM_SHARED((NP, HALF), jnp.float32),
          pltpu.SemaphoreType.DMA,
          pltpu.SemaphoreType.DMA,
          pltpu.SemaphoreType.DMA,
          pltpu.SemaphoreType.DMA,
          pltpu.SemaphoreType.DMA,
          pltpu.SemaphoreType.DMA,
          pltpu.SemaphoreType.DMA,
      ],
  )
  return f(x0, epk, ew, pidx, fwv)


def kernel(users, items, user_emb, item_emb, edge_src, edge_dst, edge_w,
           fw1, fw2, fw3, fw4):
  all_emb = jnp.concatenate([user_emb, item_emb], axis=0)
  npad = jnp.zeros((NP - NN, HALF), jnp.float32)
  x0 = jnp.concatenate(
      [all_emb[:, :HALF], npad, all_emb[:, HALF:], npad], 0)

  pad = EPAD - E0
  esrc = jnp.concatenate([edge_src.astype(jnp.int32),
                          jnp.zeros((pad,), jnp.int32)]).reshape(
                              NS * NCHUNK, CH)
  edst = jnp.concatenate([edge_dst.astype(jnp.int32),
                          jnp.zeros((pad,), jnp.int32)]).reshape(
                              NS * NCHUNK, CH)
  ew = jnp.concatenate([edge_w.astype(jnp.float32),
                        jnp.zeros((pad,), jnp.float32)])
  epk = jnp.stack([
      jnp.stack([esrc, edst], axis=1),
      jnp.stack([esrc + NP, edst], axis=1),
  ], axis=0)

  zpad = jnp.zeros((NS, CH - PB), jnp.int32)
  u2 = jnp.concatenate([users.astype(jnp.int32).reshape(NS, PB), zpad], 1)
  i2 = jnp.concatenate([(items.astype(jnp.int32) + NU).reshape(NS, PB),
                        zpad], 1)
  pidx0 = jnp.stack([u2, i2], axis=1).reshape(NS * 2, CH)
  pidx = jnp.stack([pidx0, pidx0 + NP], axis=0)

  fwv = jnp.zeros((8, 16), jnp.float32)
  fwv = fwv.at[0:4].set(
      jnp.broadcast_to(
          jnp.stack([fw1, fw2, fw3, fw4]).reshape(4, 1).astype(jnp.float32),
          (4, 16)))

  gam, _, _, _ = _run(x0, epk, ew, pidx, fwv)
  gam = gam.reshape(NC, BATCH)
  return gam[0] + gam[1]
